# Initial kernel scaffold; baseline (speedup 1.0000x reference)
import jax
import jax.numpy as jnp
from jax.experimental import pallas as pl


def kernel(x, Wq, bq, Wk, bk, Wv, bv, Wh, bh, g1kv, be1kv, g1q, be1q, Wf1, bf1, Wf2, bf2, g2, be2):
    raise NotImplementedError("write your pallas kernel here")


# speedup vs baseline: 3.4538x; 3.4538x over previous
import numpy as np
import jax
import jax.numpy as jnp
from jax.experimental import pallas as pl
from jax.experimental.pallas import tpu as pltpu

N = 32768
D = 512
H = 8
HD = D // H
DF = 4 * D
L = 2
BN = 512
NB = N // BN
_EPS = 1e-5
_FN = float(N)
_ISQRT2 = np.float32(0.7071067811865476)

_M_NP = np.kron(np.eye(H, dtype=np.float32), np.ones((HD, HD), np.float32))


def _pass_a(h_ref, g1kv_ref, be1kv_ref, g1q_ref, be1q_ref,
            wq_ref, bq_ref, wk_ref, bk_ref, wv_ref, bv_ref,
            q_ref, k_ref, v_ref, red_ref):
    h = h_ref[...]
    m = jnp.mean(h, axis=1, keepdims=True)
    c = h - m
    var = jnp.mean(c * c, axis=1, keepdims=True)
    xn = c * jax.lax.rsqrt(var + _EPS)
    src = (xn * g1kv_ref[...] + be1kv_ref[...]).astype(jnp.bfloat16)
    qry = (xn * g1q_ref[...] + be1q_ref[...]).astype(jnp.bfloat16)
    q = jnp.dot(qry, wq_ref[...], preferred_element_type=jnp.float32) + bq_ref[...]
    k = jnp.dot(src, wk_ref[...], preferred_element_type=jnp.float32) + bk_ref[...]
    v = jnp.dot(src, wv_ref[...], preferred_element_type=jnp.float32) + bv_ref[...]
    q_ref[...] = q.astype(jnp.bfloat16)
    k_ref[...] = k.astype(jnp.bfloat16)
    v_ref[...] = v.astype(jnp.bfloat16)
    z = jnp.zeros((1, D), jnp.float32)
    sums = jnp.concatenate([
        jnp.sum(k * v, axis=0, keepdims=True),
        jnp.sum(k, axis=0, keepdims=True),
        jnp.sum(q * q, axis=0, keepdims=True),
        jnp.sum(k * k, axis=0, keepdims=True),
        z, z, z, z], axis=0)
    red_ref[...] = sums.reshape(1, 8, D)


def _pass_b(q_ref, v_ref, h_ref, rp_ref, m_ref,
            wh_ref, bh_ref, g2_ref, be2_ref,
            wf1_ref, bf1_ref, wf2_ref, bf2_ref, o_ref):
    red = jnp.sum(rp_ref[...], axis=0)
    q2s = jnp.sum(red[2:3, :])
    k2s = jnp.sum(red[3:4, :])
    rsc = jax.lax.rsqrt(q2s * k2s)
    srow = red[0:1, :] * rsc
    krow = red[1:2, :] * rsc
    q = q_ref[...].astype(jnp.float32)
    v = v_ref[...].astype(jnp.float32)
    num = q * srow + v * _FN
    t = (q * krow).astype(jnp.bfloat16)
    den = jnp.dot(t, m_ref[...], preferred_element_type=jnp.float32) + _FN
    attn = (num / den).astype(jnp.bfloat16)
    hp = (jnp.dot(attn, wh_ref[...], preferred_element_type=jnp.float32)
          + bh_ref[...] + h_ref[...])
    mm = jnp.mean(hp, axis=1, keepdims=True)
    c = hp - mm
    var = jnp.mean(c * c, axis=1, keepdims=True)
    zn = (c * jax.lax.rsqrt(var + _EPS) * g2_ref[...] + be2_ref[...]).astype(jnp.bfloat16)
    f1 = jnp.dot(zn, wf1_ref[...], preferred_element_type=jnp.float32) + bf1_ref[...]
    f1 = (0.5 * f1 * (1.0 + jax.lax.erf(f1 * _ISQRT2))).astype(jnp.bfloat16)
    o_ref[...] = (jnp.dot(f1, wf2_ref[...], preferred_element_type=jnp.float32)
                  + bf2_ref[...] + hp)


def _row_spec():
    return pl.BlockSpec((1, D), lambda n: (0, 0))


def _mat_spec(shape):
    return pl.BlockSpec(shape, lambda n: (0, 0))


def _blk_spec():
    return pl.BlockSpec((BN, D), lambda n: (n, 0))


def _layer(h, wq, bq, wk, bk, wv, bv, wh, bh,
           g1kv, be1kv, g1q, be1q, wf1, bf1, wf2, bf2, g2, be2, mblk):
    row = lambda a: a.reshape(1, -1)
    bf = lambda a: a.astype(jnp.bfloat16)
    grid = (NB,)
    params = pltpu.CompilerParams(dimension_semantics=("parallel",))

    q, k, v, rp = pl.pallas_call(
        _pass_a,
        grid=grid,
        in_specs=[
            _blk_spec(),
            _row_spec(), _row_spec(), _row_spec(), _row_spec(),
            _mat_spec((D, D)), _row_spec(),
            _mat_spec((D, D)), _row_spec(),
            _mat_spec((D, D)), _row_spec(),
        ],
        out_specs=[
            _blk_spec(), _blk_spec(), _blk_spec(),
            pl.BlockSpec((1, 8, D), lambda n: (n, 0, 0)),
        ],
        out_shape=[
            jax.ShapeDtypeStruct((N, D), jnp.bfloat16),
            jax.ShapeDtypeStruct((N, D), jnp.bfloat16),
            jax.ShapeDtypeStruct((N, D), jnp.bfloat16),
            jax.ShapeDtypeStruct((NB, 8, D), jnp.float32),
        ],
        compiler_params=params,
    )(h, row(g1kv), row(be1kv), row(g1q), row(be1q),
      bf(wq), row(bq), bf(wk), row(bk), bf(wv), row(bv))

    out = pl.pallas_call(
        _pass_b,
        grid=grid,
        in_specs=[
            _blk_spec(), _blk_spec(), _blk_spec(),
            pl.BlockSpec((NB, 8, D), lambda n: (0, 0, 0)),
            _mat_spec((D, D)),
            _mat_spec((D, D)), _row_spec(),
            _row_spec(), _row_spec(),
            _mat_spec((D, DF)), pl.BlockSpec((1, DF), lambda n: (0, 0)),
            _mat_spec((DF, D)), _row_spec(),
        ],
        out_specs=_blk_spec(),
        out_shape=jax.ShapeDtypeStruct((N, D), jnp.float32),
        compiler_params=params,
    )(q, v, h, rp, mblk,
      bf(wh), row(bh), row(g2), row(be2),
      bf(wf1), row(bf1), bf(wf2), row(bf2))
    return out


def kernel(x, Wq, bq, Wk, bk, Wv, bv, Wh, bh, g1kv, be1kv, g1q, be1q,
           Wf1, bf1, Wf2, bf2, g2, be2):
    mblk = jnp.asarray(_M_NP, jnp.bfloat16)
    h = x
    for i in range(L):
        h = _layer(h, Wq[i], bq[i], Wk[i], bk[i], Wv[i], bv[i], Wh[i], bh[i],
                   g1kv[i], be1kv[i], g1q[i], be1q[i],
                   Wf1[i], bf1[i], Wf2[i], bf2[i], g2[i], be2[i], mblk)
    return h
